# baseline (speedup 1.0000x reference)
import jax
import jax.numpy as jnp
from jax.experimental import pallas as pl
from jax.experimental.pallas import tpu as pltpu

B, T, N, FIN, FOUT = 4, 8, 512, 128, 64
BT = B * T
C = 4
NCH = BT // C
NSLOT = 3
CPB = T // C


def _gat_body(inp_hbm, adj_hbm, w_ref, a_ref, out_hbm,
              xbuf, obuf, adj_vmem, in_sem, out_sem, adj_sem):
    def in_copy(c):
        slot = c % NSLOT
        return pltpu.make_async_copy(
            inp_hbm.at[c // CPB, pl.ds((c % CPB) * C, C)],
            xbuf.at[slot], in_sem.at[slot])

    def out_copy(c):
        slot = c % NSLOT
        return pltpu.make_async_copy(
            obuf.at[slot],
            out_hbm.at[c // CPB, pl.ds((c % CPB) * C, C)],
            out_sem.at[slot])

    adj_copy = pltpu.make_async_copy(adj_hbm, adj_vmem, adj_sem)
    adj_copy.start()
    for c in range(NSLOT):
        in_copy(c).start()

    wb = w_ref[...].astype(jnp.bfloat16)
    a1 = a_ref[:FOUT, :]
    a2 = a_ref[FOUT:, :]
    adj_copy.wait()
    mask = (adj_vmem[...] > 0).astype(jnp.bfloat16)

    for c in range(NCH):
        slot = c % NSLOT
        in_copy(c).wait()
        if c >= NSLOT:
            out_copy(c - NSLOT).wait()
        for k in range(C):
            xb = xbuf[slot, k].astype(jnp.bfloat16)
            ht = jax.lax.dot_general(wb, xb, (((0,), (1,)), ((), ())),
                                     preferred_element_type=jnp.float32)
            h2t = jnp.dot(ht.astype(jnp.bfloat16), mask,
                          preferred_element_type=jnp.float32)
            score = (jnp.sum(ht * a1, axis=0)
                     + jnp.sum(h2t * a2, axis=0))
            colsum = jnp.sum(ht, axis=1)
            obuf[slot, k] = jnp.maximum(score[:, None] * colsum[None, :], 0.0)
        out_copy(c).start()
        if c + NSLOT < NCH:
            in_copy(c + NSLOT).start()

    for c in range(NCH - NSLOT, NCH):
        out_copy(c).wait()


def kernel(inp, adj, W, a):
    f = W.shape[1]
    return pl.pallas_call(
        _gat_body,
        in_specs=[
            pl.BlockSpec(memory_space=pl.ANY),
            pl.BlockSpec(memory_space=pl.ANY),
            pl.BlockSpec(memory_space=pltpu.MemorySpace.VMEM),
            pl.BlockSpec(memory_space=pltpu.MemorySpace.VMEM),
        ],
        out_specs=pl.BlockSpec(memory_space=pl.ANY),
        out_shape=jax.ShapeDtypeStruct((B, T, N, f), jnp.float32),
        scratch_shapes=[
            pltpu.VMEM((NSLOT, C, N, FIN), jnp.float32),
            pltpu.VMEM((NSLOT, C, N, FOUT), jnp.float32),
            pltpu.VMEM((N, N), jnp.float32),
            pltpu.SemaphoreType.DMA((NSLOT,)),
            pltpu.SemaphoreType.DMA((NSLOT,)),
            pltpu.SemaphoreType.DMA,
        ],
    )(inp, adj, W, a)


# speedup vs baseline: 1.6909x; 1.6909x over previous
import jax
import jax.numpy as jnp
from jax.experimental import pallas as pl
from jax.experimental.pallas import tpu as pltpu

B, T, N, FIN, FOUT = 4, 8, 512, 128, 64
BT = B * T
C = 4
NCH = BT // C
NSLOT = 3
CPB = T // C


def _gat_body(inp_hbm, adj_hbm, w_ref, a_ref, score_hbm, s_hbm,
              xbuf, score_buf, s_buf, adj_vmem,
              in_sem, out_sem, adj_sem):
    def in_copy(c):
        slot = c % NSLOT
        return pltpu.make_async_copy(
            inp_hbm.at[c // CPB, pl.ds((c % CPB) * C, C)],
            xbuf.at[slot], in_sem.at[slot])

    adj_copy = pltpu.make_async_copy(adj_hbm, adj_vmem, adj_sem)
    score_copy = pltpu.make_async_copy(score_buf, score_hbm, out_sem.at[0])
    s_copy = pltpu.make_async_copy(s_buf, s_hbm, out_sem.at[1])

    adj_copy.start()
    for c in range(NSLOT):
        in_copy(c).start()

    wb = w_ref[...].astype(jnp.bfloat16)
    a1 = a_ref[:FOUT, :]
    a2 = a_ref[FOUT:, :]
    adj_copy.wait()
    mask = (adj_vmem[...] > 0).astype(jnp.bfloat16)

    for c in range(NCH):
        slot = c % NSLOT
        b = c // CPB
        t0 = (c % CPB) * C
        in_copy(c).wait()
        hts = []
        for k in range(C):
            xb = xbuf[slot, k].astype(jnp.bfloat16)
            hts.append(jax.lax.dot_general(
                wb, xb, (((0,), (1,)), ((), ())),
                preferred_element_type=jnp.float32))
        hts_b = jnp.concatenate([h.astype(jnp.bfloat16) for h in hts],
                                axis=0)
        h2ts = jnp.dot(hts_b, mask, preferred_element_type=jnp.float32)
        for k in range(C):
            ht = hts[k]
            h2t = h2ts[k * FOUT:(k + 1) * FOUT, :]
            score = (jnp.sum(ht * a1, axis=0)
                     + jnp.sum(h2t * a2, axis=0))
            score_buf[b, t0 + k] = score
            s_buf[b, t0 + k] = jnp.sum(ht, axis=1)
        if c + NSLOT < NCH:
            in_copy(c + NSLOT).start()

    score_copy.start()
    s_copy.start()
    score_copy.wait()
    s_copy.wait()


def kernel(inp, adj, W, a):
    f = W.shape[1]
    score, s = pl.pallas_call(
        _gat_body,
        in_specs=[
            pl.BlockSpec(memory_space=pl.ANY),
            pl.BlockSpec(memory_space=pl.ANY),
            pl.BlockSpec(memory_space=pltpu.MemorySpace.VMEM),
            pl.BlockSpec(memory_space=pltpu.MemorySpace.VMEM),
        ],
        out_specs=[
            pl.BlockSpec(memory_space=pl.ANY),
            pl.BlockSpec(memory_space=pl.ANY),
        ],
        out_shape=[
            jax.ShapeDtypeStruct((B, T, N), jnp.float32),
            jax.ShapeDtypeStruct((B, T, f), jnp.float32),
        ],
        scratch_shapes=[
            pltpu.VMEM((NSLOT, C, N, FIN), jnp.float32),
            pltpu.VMEM((B, T, N), jnp.float32),
            pltpu.VMEM((B, T, FOUT), jnp.float32),
            pltpu.VMEM((N, N), jnp.float32),
            pltpu.SemaphoreType.DMA((NSLOT,)),
            pltpu.SemaphoreType.DMA((2,)),
            pltpu.SemaphoreType.DMA,
        ],
    )(inp, adj, W, a)

    return jnp.maximum(score[..., None] * s[..., None, :], 0.0)
